# baseline (speedup 1.0000x reference)
import functools
import math

import jax
import jax.numpy as jnp
from jax.experimental import pallas as pl
from jax.experimental.pallas import tpu as pltpu
from jax.experimental.pallas import tpu_sc as plsc

DIM = 384
NUM_HEADS = 12
HD = DIM // NUM_HEADS
KER = 7
NUM_EXPERTS = 8
HID = 768
B, H, W = 2, 32, 32
T = B * H * W
GROUP = 4
NG = H // GROUP
SLAB = GROUP + KER - 1
QBLK = GROUP * W
KVBLK = SLAB * W
SCALE = HD ** -0.5
BF = jnp.bfloat16

SLOTS = 2 * T
BLK = 256
NBLK = 23
NBLK_PAD = 24
NBUF = NBLK * BLK
NW = 32
CHUNK = SLOTS // NW


def _gelu(v):
    return 0.5 * v * (1.0 + jax.lax.erf(v / math.sqrt(2.0)))


def _dotf32(a, b, trans_b=False):
    dn = (((1,), (1,)), ((), ())) if trans_b else (((1,), (0,)), ((), ()))
    return jax.lax.dot_general(a, b, dn, preferred_element_type=jnp.float32)


def _k1_body(x_ref, g_ref, b_ref, wqkv_ref, bqkv_ref, o_ref):
    x = x_ref[...]
    m = jnp.mean(x, axis=-1, keepdims=True)
    v = jnp.mean((x - m) ** 2, axis=-1, keepdims=True)
    h = (x - m) * jax.lax.rsqrt(v + 1e-5) * g_ref[...] + b_ref[...]
    acc = _dotf32(h.astype(BF), wqkv_ref[...]) + bqkv_ref[...]
    o_ref[...] = acc.astype(BF)


def _k1(xf, ln1_g, ln1_b, W_qkv_bf, b_qkv):
    blk = 512
    return pl.pallas_call(
        _k1_body,
        grid=(T // blk,),
        in_specs=[
            pl.BlockSpec((blk, DIM), lambda i: (i, 0)),
            pl.BlockSpec((DIM,), lambda i: (0,)),
            pl.BlockSpec((DIM,), lambda i: (0,)),
            pl.BlockSpec((DIM, 3 * DIM), lambda i: (0, 0)),
            pl.BlockSpec((3 * DIM,), lambda i: (0,)),
        ],
        out_specs=pl.BlockSpec((blk, 3 * DIM), lambda i: (i, 0)),
        out_shape=jax.ShapeDtypeStruct((T, 3 * DIM), BF),
    )(xf, ln1_g, ln1_b, W_qkv_bf, b_qkv)


def _k2_body(bias_ref, q_ref, kv_ref, o_ref):
    g = pl.program_id(1)
    start = jnp.clip(GROUP * g - (KER // 2), 0, H - SLAB) * W
    w_q = jax.lax.broadcasted_iota(jnp.int32, (QBLK, KVBLK), 0) & (W - 1)
    w_k = jax.lax.broadcasted_iota(jnp.int32, (QBLK, KVBLK), 1) & (W - 1)
    kj = w_k - jnp.clip(w_q - KER // 2, 0, W - KER)
    maskw = jnp.where((kj >= 0) & (kj < KER), 0.0, -1e9)
    for n in range(NUM_HEADS):
        q = q_ref[0, :, n * HD:(n + 1) * HD]
        ks = kv_ref[0, pl.ds(start, KVBLK), DIM + n * HD:DIM + (n + 1) * HD]
        vs = kv_ref[0, pl.ds(start, KVBLK), 2 * DIM + n * HD:2 * DIM + (n + 1) * HD]
        bias_n = jnp.concatenate(
            [bias_ref[0, :, j, :, n].reshape(QBLK, W) for j in range(SLAB)], axis=1)
        s = (_dotf32(q, ks, trans_b=True) * SCALE
             + bias_n.astype(jnp.float32) + maskw)
        p = jnp.exp(s)
        acc = _dotf32(p.astype(BF), vs)
        o_ref[0, :, n * HD:(n + 1) * HD] = acc / jnp.sum(p, axis=-1, keepdims=True)


def _k2(bias, qkv3):
    return pl.pallas_call(
        _k2_body,
        grid=(B, NG),
        in_specs=[
            pl.BlockSpec((1, GROUP, SLAB, W, NUM_HEADS, W),
                         lambda b, g: (g, 0, 0, 0, 0, 0)),
            pl.BlockSpec((1, QBLK, 3 * DIM), lambda b, g: (b, g, 0)),
            pl.BlockSpec((1, H * W, 3 * DIM), lambda b, g: (b, 0, 0)),
        ],
        out_specs=pl.BlockSpec((1, QBLK, DIM), lambda b, g: (b, g, 0)),
        out_shape=jax.ShapeDtypeStruct((B, H * W, DIM), jnp.float32),
    )(bias, qkv3, qkv3)


def _bias_tables():
    import numpy as np
    NR = 2 * KER - 1
    starts = np.clip(np.arange(H) - KER // 2, 0, H - KER)
    S = np.clip(GROUP * np.arange(NG) - KER // 2, 0, H - SLAB)
    R = GROUP * np.arange(NG)[:, None] + np.arange(GROUP)[None, :]
    j = np.arange(SLAB)
    ki = S[:, None, None] + j[None, None, :] - starts[R][:, :, None]
    valid_h = (ki >= 0) & (ki < KER)
    rel_h = S[:, None, None] + j[None, None, :] - R[:, :, None] + (KER - 1)
    oh_h = np.eye(NR, dtype=np.float32)[np.clip(rel_h, 0, NR - 1)] * valid_h[..., None]
    maskh = np.where(valid_h, 0.0, -1e9).astype(np.float32)
    w2 = np.arange(W)
    kj = w2[None, :] - starts[:, None]
    valid_w = (kj >= 0) & (kj < KER)
    rel_w = w2[None, :] - np.arange(W)[:, None] + (KER - 1)
    oh_w = np.eye(NR, dtype=np.float32)[np.clip(rel_w, 0, NR - 1)] * valid_w[..., None]
    return oh_h, oh_w, maskh


_OH_H, _OH_W, _MASKH = _bias_tables()


def _make_bias(rpb):
    t = jnp.einsum('grja,nab->grjnb', jnp.asarray(_OH_H), rpb)
    t = t + jnp.asarray(_MASKH)[:, :, :, None, None]
    bias = jnp.einsum('grjnb,wvb->grjwnv', t, jnp.asarray(_OH_W))
    return bias.astype(BF)


def _k3_body(attn_ref, x_ref, wp_ref, bp_ref, g2_ref, b2_ref, wr_ref, br_ref,
             ws1_ref, bs1_ref, ws2_ref, bs2_ref,
             y_ref, base_ref, comb_ref):
    x2 = x_ref[...] + _dotf32(attn_ref[...].astype(BF), wp_ref[...]) + bp_ref[...]
    m = jnp.mean(x2, axis=-1, keepdims=True)
    v = jnp.mean((x2 - m) ** 2, axis=-1, keepdims=True)
    y = (x2 - m) * jax.lax.rsqrt(v + 1e-5) * g2_ref[...] + b2_ref[...]
    y_ref[...] = y
    yb = y.astype(BF)
    logits = _dotf32(y, wr_ref[...]) + br_ref[...]
    mx = jnp.max(logits, axis=-1, keepdims=True)
    eg = jnp.exp(logits - mx)
    gates = eg / jnp.sum(eg, axis=-1, keepdims=True)
    iota = jax.lax.broadcasted_iota(jnp.int32, gates.shape, 1)
    v1 = jnp.max(gates, axis=-1, keepdims=True)
    i1 = jnp.min(jnp.where(gates == v1, iota, NUM_EXPERTS), axis=-1, keepdims=True)
    m1 = iota == i1
    g2nd = jnp.where(m1, -1.0, gates)
    v2 = jnp.max(g2nd, axis=-1, keepdims=True)
    i2 = jnp.min(jnp.where(g2nd == v2, iota, NUM_EXPERTS), axis=-1, keepdims=True)
    m2 = iota == i2
    denom = v1 + v2
    comb_ref[...] = (jnp.where(m1, v1, 0.0) + jnp.where(m2, v2, 0.0)) / denom
    hsh = _gelu(_dotf32(yb, ws1_ref[...]) + bs1_ref[...])
    shared = _dotf32(hsh.astype(BF), ws2_ref[...]) + bs2_ref[...]
    base_ref[...] = x2 + shared


def _k3(attn, xf, Wp_bf, b_proj, ln2_g, ln2_b, W_r, b_r, Ws1_bf, b_s1, Ws2_bf, b_s2):
    blk = 512
    return pl.pallas_call(
        _k3_body,
        grid=(T // blk,),
        in_specs=[
            pl.BlockSpec((blk, DIM), lambda i: (i, 0)),
            pl.BlockSpec((blk, DIM), lambda i: (i, 0)),
            pl.BlockSpec((DIM, DIM), lambda i: (0, 0)),
            pl.BlockSpec((DIM,), lambda i: (0,)),
            pl.BlockSpec((DIM,), lambda i: (0,)),
            pl.BlockSpec((DIM,), lambda i: (0,)),
            pl.BlockSpec((DIM, NUM_EXPERTS), lambda i: (0, 0)),
            pl.BlockSpec((NUM_EXPERTS,), lambda i: (0,)),
            pl.BlockSpec((DIM, HID), lambda i: (0, 0)),
            pl.BlockSpec((HID,), lambda i: (0,)),
            pl.BlockSpec((HID, DIM), lambda i: (0, 0)),
            pl.BlockSpec((DIM,), lambda i: (0,)),
        ],
        out_specs=[
            pl.BlockSpec((blk, DIM), lambda i: (i, 0)),
            pl.BlockSpec((blk, DIM), lambda i: (i, 0)),
            pl.BlockSpec((blk, NUM_EXPERTS), lambda i: (i, 0)),
        ],
        out_shape=[
            jax.ShapeDtypeStruct((T, DIM), jnp.float32),
            jax.ShapeDtypeStruct((T, DIM), jnp.float32),
            jax.ShapeDtypeStruct((T, NUM_EXPERTS), jnp.float32),
        ],
    )(attn, xf, Wp_bf, b_proj, ln2_g, ln2_b, W_r, b_r, Ws1_bf, b_s1, Ws2_bf, b_s2)


def _cumsum_lanes(x, n):
    d = 1
    while d < n:
        shifted = jnp.concatenate([jnp.zeros((x.shape[0], d), x.dtype), x[:, :n - d]],
                                  axis=1)
        x = x + shifted
        d *= 2
    return x


def _k3b_body(comb_ref, dest_ref, w_ref, bexp_ref):
    c = jnp.transpose(comb_ref[...], (1, 0))
    m = c > 0.0
    iot = jax.lax.broadcasted_iota(jnp.int32, (NUM_EXPERTS, T), 0)
    e_sm = jnp.min(jnp.where(m, iot, NUM_EXPERTS), axis=0, keepdims=True)
    e_bg = jnp.max(jnp.where(m, iot, -1), axis=0, keepdims=True)
    oh1 = (iot == e_sm).astype(jnp.float32)
    oh2 = (iot == e_bg).astype(jnp.float32)
    w1 = jnp.sum(c * oh1, axis=0, keepdims=True)
    w2 = jnp.sum(c * oh2, axis=0, keepdims=True)
    cs1 = _cumsum_lanes(oh1, T)
    cs2 = _cumsum_lanes(oh2, T)
    tot1 = cs1[:, T - 1:T]
    tot2 = cs2[:, T - 1:T]
    counts = tot1 + tot2
    nblk = jnp.floor((counts + (BLK - 1)) / BLK)
    le = (jax.lax.broadcasted_iota(jnp.int32, (NUM_EXPERTS, NUM_EXPERTS), 1)
          <= jax.lax.broadcasted_iota(jnp.int32, (NUM_EXPERTS, NUM_EXPERTS), 0))
    c_incl = _dotf32(le.astype(jnp.float32), nblk)
    offs = BLK * (c_incl - nblk)
    dest1 = jnp.sum((offs - 1.0) * oh1 + cs1 * oh1, axis=0, keepdims=True)
    dest2 = jnp.sum((offs + tot1 - 1.0) * oh2 + cs2 * oh2, axis=0, keepdims=True)
    zero6 = jnp.zeros((6, T), jnp.float32)
    dest_ref[...] = jnp.concatenate([dest1, dest2, zero6], axis=0).astype(jnp.int32)
    w_ref[...] = jnp.concatenate([w1, w2, zero6], axis=0)
    jot = jax.lax.broadcasted_iota(jnp.int32, (NBLK_PAD, NUM_EXPERTS), 0).astype(jnp.float32)
    cT = jnp.transpose(c_incl, (1, 0))
    bexp = jnp.sum((jot >= cT).astype(jnp.float32), axis=1, keepdims=True)
    bexp_ref[...] = jnp.clip(bexp, 0.0, NUM_EXPERTS - 1.0).astype(jnp.int32)


def _k3b(comb):
    full = lambda *shape: pl.BlockSpec(shape, lambda: tuple(0 for _ in shape))
    return pl.pallas_call(
        _k3b_body,
        grid=(),
        in_specs=[full(T, NUM_EXPERTS)],
        out_specs=[full(8, T), full(8, T), full(NBLK_PAD, 1)],
        out_shape=[
            jax.ShapeDtypeStruct((8, T), jnp.int32),
            jax.ShapeDtypeStruct((8, T), jnp.float32),
            jax.ShapeDtypeStruct((NBLK_PAD, 1), jnp.int32),
        ],
    )(comb)


def _sc_dispatch(y32, dest32):
    mesh = plsc.VectorSubcoreMesh(core_axis_name="c", subcore_axis_name="s")

    @functools.partial(
        pl.kernel,
        out_type=jax.ShapeDtypeStruct((NBUF, DIM), jnp.float32),
        mesh=mesh,
        scratch_types=[
            pltpu.VMEM((CHUNK,), jnp.int32),
            pltpu.VMEM((CHUNK, DIM), jnp.float32),
            pltpu.SemaphoreType.DMA,
        ],
    )
    def disp(y_hbm, d_hbm, ybuf_hbm, idx_v, rows_v, sem):
        wid = jax.lax.axis_index("s") * 2 + jax.lax.axis_index("c")
        base = wid * CHUNK
        tok = jax.lax.rem(base, T)
        pltpu.sync_copy(d_hbm.at[wid], idx_v)
        pltpu.sync_copy(y_hbm.at[pl.ds(tok, CHUNK)], rows_v)
        pltpu.async_copy(rows_v, ybuf_hbm.at[idx_v], sem).wait()

    return disp(y32, dest32)


def _sc_combine(obuf, dest32):
    mesh = plsc.VectorSubcoreMesh(core_axis_name="c", subcore_axis_name="s")

    @functools.partial(
        pl.kernel,
        out_type=jax.ShapeDtypeStruct((SLOTS, DIM), jnp.float32),
        mesh=mesh,
        scratch_types=[
            pltpu.VMEM((CHUNK,), jnp.int32),
            pltpu.VMEM((CHUNK, DIM), jnp.float32),
            pltpu.SemaphoreType.DMA,
        ],
    )
    def comb_k(o_hbm, d_hbm, out_hbm, idx_v, rows_v, sem):
        wid = jax.lax.axis_index("s") * 2 + jax.lax.axis_index("c")
        base = wid * CHUNK
        pltpu.sync_copy(d_hbm.at[wid], idx_v)
        pltpu.async_copy(o_hbm.at[idx_v], rows_v, sem).wait()
        pltpu.sync_copy(rows_v, out_hbm.at[pl.ds(base, CHUNK)])

    return comb_k(obuf, dest32)


def _k6_body(bexp_ref, y_ref, w1_ref, b1_ref, w2_ref, b2_ref, o_ref):
    h = _gelu(_dotf32(y_ref[...].astype(BF), w1_ref[0]) + b1_ref[0])
    o_ref[...] = _dotf32(h.astype(BF), w2_ref[0]) + b2_ref[0]


def _k6(bexp, ybuf, We1_bf, b_e1r, We2_bf, b_e2r):
    grid_spec = pltpu.PrefetchScalarGridSpec(
        num_scalar_prefetch=1,
        grid=(NBLK,),
        in_specs=[
            pl.BlockSpec((BLK, DIM), lambda j, be: (j, 0)),
            pl.BlockSpec((1, DIM, HID), lambda j, be: (be[j], 0, 0)),
            pl.BlockSpec((1, 1, HID), lambda j, be: (be[j], 0, 0)),
            pl.BlockSpec((1, HID, DIM), lambda j, be: (be[j], 0, 0)),
            pl.BlockSpec((1, 1, DIM), lambda j, be: (be[j], 0, 0)),
        ],
        out_specs=pl.BlockSpec((BLK, DIM), lambda j, be: (j, 0)),
    )
    return pl.pallas_call(
        _k6_body,
        grid_spec=grid_spec,
        out_shape=jax.ShapeDtypeStruct((NBUF, DIM), jnp.float32),
    )(bexp, ybuf, We1_bf, b_e1r, We2_bf, b_e2r)


def _k5_body(base_ref, ot_ref, ob_ref, w_ref, o_ref):
    wp = jnp.transpose(w_ref[0:2, :], (1, 0))
    o_ref[...] = (base_ref[...] + wp[:, 0:1] * ot_ref[...]
                  + wp[:, 1:2] * ob_ref[...])


def _k5(base, out_slot, w8):
    blk = 512
    nb = T // blk
    return pl.pallas_call(
        _k5_body,
        grid=(nb,),
        in_specs=[
            pl.BlockSpec((blk, DIM), lambda i: (i, 0)),
            pl.BlockSpec((blk, DIM), lambda i: (i, 0)),
            pl.BlockSpec((blk, DIM), lambda i, _nb=nb: (i + _nb, 0)),
            pl.BlockSpec((8, blk), lambda i: (0, i)),
        ],
        out_specs=pl.BlockSpec((blk, DIM), lambda i: (i, 0)),
        out_shape=jax.ShapeDtypeStruct((T, DIM), jnp.float32),
    )(base, out_slot, out_slot, w8)


def kernel(x, ln1_g, ln1_b, ln2_g, ln2_b, W_qkv, b_qkv, rpb, W_proj, b_proj,
           W_r, b_r, W_e1, b_e1, W_e2, b_e2, W_s1, b_s1, W_s2, b_s2):
    xf = x.reshape(T, DIM)
    qkv = _k1(xf, ln1_g, ln1_b, W_qkv.astype(BF), b_qkv)
    qkv3 = qkv.reshape(B, H * W, 3 * DIM)
    bias = _make_bias(rpb)
    attn = _k2(bias, qkv3).reshape(T, DIM)
    y32, base, comb = _k3(attn, xf, W_proj.astype(BF), b_proj, ln2_g, ln2_b,
                          W_r, b_r, W_s1.astype(BF), b_s1, W_s2.astype(BF), b_s2)
    dest8, w8, bexp = _k3b(comb)
    dest32 = dest8[:2].reshape(NW, CHUNK)
    ybuf = _sc_dispatch(y32, dest32)
    obuf = _k6(bexp.reshape(NBLK_PAD), ybuf, W_e1.astype(BF),
               b_e1.reshape(NUM_EXPERTS, 1, HID), W_e2.astype(BF),
               b_e2.reshape(NUM_EXPERTS, 1, DIM))
    out_slot = _sc_combine(obuf, dest32)
    out = _k5(base, out_slot, w8)
    return out.reshape(B, H, W, DIM)


# speedup vs baseline: 1.0546x; 1.0546x over previous
import functools
import math

import jax
import jax.numpy as jnp
from jax.experimental import pallas as pl
from jax.experimental.pallas import tpu as pltpu
from jax.experimental.pallas import tpu_sc as plsc

DIM = 384
NUM_HEADS = 12
HD = DIM // NUM_HEADS
KER = 7
NUM_EXPERTS = 8
HID = 768
B, H, W = 2, 32, 32
T = B * H * W
GROUP = 4
NG = H // GROUP
SLAB = GROUP + KER - 1
QBLK = GROUP * W
KVBLK = SLAB * W
SCALE = HD ** -0.5
BF = jnp.bfloat16

SLOTS = 2 * T
BLK = 256
NBLK = 23
NBLK_PAD = 24
NBUF = NBLK * BLK
NW = 32
CHUNK = SLOTS // NW


def _gelu(v):
    return 0.5 * v * (1.0 + jax.lax.erf(v / math.sqrt(2.0)))


def _dotf32(a, b, trans_b=False):
    dn = (((1,), (1,)), ((), ())) if trans_b else (((1,), (0,)), ((), ()))
    return jax.lax.dot_general(a, b, dn, preferred_element_type=jnp.float32)


def _k1_body(x_ref, g_ref, b_ref, wqkv_ref, bqkv_ref, o_ref):
    x = x_ref[...]
    m = jnp.mean(x, axis=-1, keepdims=True)
    v = jnp.mean((x - m) ** 2, axis=-1, keepdims=True)
    h = (x - m) * jax.lax.rsqrt(v + 1e-5) * g_ref[...] + b_ref[...]
    acc = _dotf32(h.astype(BF), wqkv_ref[...]) + bqkv_ref[...]
    o_ref[...] = acc.astype(BF)


def _k1(xf, ln1_g, ln1_b, W_qkv_bf, b_qkv):
    blk = 512
    return pl.pallas_call(
        _k1_body,
        grid=(T // blk,),
        in_specs=[
            pl.BlockSpec((blk, DIM), lambda i: (i, 0)),
            pl.BlockSpec((DIM,), lambda i: (0,)),
            pl.BlockSpec((DIM,), lambda i: (0,)),
            pl.BlockSpec((DIM, 3 * DIM), lambda i: (0, 0)),
            pl.BlockSpec((3 * DIM,), lambda i: (0,)),
        ],
        out_specs=pl.BlockSpec((blk, 3 * DIM), lambda i: (i, 0)),
        out_shape=jax.ShapeDtypeStruct((T, 3 * DIM), BF),
    )(xf, ln1_g, ln1_b, W_qkv_bf, b_qkv)


def _k2_body(bias_ref, q_ref, kv_ref, o_ref):
    g = pl.program_id(1)
    start = jnp.clip(GROUP * g - (KER // 2), 0, H - SLAB) * W
    w_q = jax.lax.broadcasted_iota(jnp.int32, (QBLK, KVBLK), 0) & (W - 1)
    w_k = jax.lax.broadcasted_iota(jnp.int32, (QBLK, KVBLK), 1) & (W - 1)
    kj = w_k - jnp.clip(w_q - KER // 2, 0, W - KER)
    maskw = jnp.where((kj >= 0) & (kj < KER), 0.0, -1e9)
    for n in range(NUM_HEADS):
        q = q_ref[0, :, n * HD:(n + 1) * HD]
        ks = kv_ref[0, pl.ds(start, KVBLK), DIM + n * HD:DIM + (n + 1) * HD]
        vs = kv_ref[0, pl.ds(start, KVBLK), 2 * DIM + n * HD:2 * DIM + (n + 1) * HD]
        bias_n = jnp.concatenate(
            [bias_ref[0, :, j, :, n].reshape(QBLK, W) for j in range(SLAB)], axis=1)
        s = _dotf32(q, ks, trans_b=True) * SCALE + bias_n + maskw
        p = jnp.exp(s)
        acc = _dotf32(p.astype(BF), vs)
        o_ref[0, :, n * HD:(n + 1) * HD] = acc / jnp.sum(p, axis=-1, keepdims=True)


def _k2(bias, qkv3):
    return pl.pallas_call(
        _k2_body,
        grid=(B, NG),
        in_specs=[
            pl.BlockSpec((1, GROUP, SLAB, W, NUM_HEADS, W),
                         lambda b, g: (g, 0, 0, 0, 0, 0)),
            pl.BlockSpec((1, QBLK, 3 * DIM), lambda b, g: (b, g, 0)),
            pl.BlockSpec((1, H * W, 3 * DIM), lambda b, g: (b, 0, 0)),
        ],
        out_specs=pl.BlockSpec((1, QBLK, DIM), lambda b, g: (b, g, 0)),
        out_shape=jax.ShapeDtypeStruct((B, H * W, DIM), jnp.float32),
    )(bias, qkv3, qkv3)


def _bias_tables():
    import numpy as np
    NR = 2 * KER - 1
    starts = np.clip(np.arange(H) - KER // 2, 0, H - KER)
    S = np.clip(GROUP * np.arange(NG) - KER // 2, 0, H - SLAB)
    R = GROUP * np.arange(NG)[:, None] + np.arange(GROUP)[None, :]
    j = np.arange(SLAB)
    ki = S[:, None, None] + j[None, None, :] - starts[R][:, :, None]
    valid_h = (ki >= 0) & (ki < KER)
    rel_h = S[:, None, None] + j[None, None, :] - R[:, :, None] + (KER - 1)
    oh_h = np.eye(NR, dtype=np.float32)[np.clip(rel_h, 0, NR - 1)] * valid_h[..., None]
    maskh = np.where(valid_h, 0.0, -1e9).astype(np.float32)
    w2 = np.arange(W)
    kj = w2[None, :] - starts[:, None]
    valid_w = (kj >= 0) & (kj < KER)
    rel_w = w2[None, :] - np.arange(W)[:, None] + (KER - 1)
    oh_w = np.eye(NR, dtype=np.float32)[np.clip(rel_w, 0, NR - 1)] * valid_w[..., None]
    return oh_h, oh_w, maskh


_OH_H, _OH_W, _MASKH = _bias_tables()


def _make_bias(rpb):
    t = jnp.einsum('grja,nab->grjnb', jnp.asarray(_OH_H), rpb)
    t = t + jnp.asarray(_MASKH)[:, :, :, None, None]
    bias = jnp.einsum('grjnb,wvb->grjwnv', t, jnp.asarray(_OH_W))
    return bias


def _k3_body(attn_ref, x_ref, wp_ref, bp_ref, g2_ref, b2_ref, wr_ref, br_ref,
             ws1_ref, bs1_ref, ws2_ref, bs2_ref,
             y_ref, base_ref, dest_ref, w_ref, bexp_ref):
    x2 = x_ref[...] + _dotf32(attn_ref[...].astype(BF), wp_ref[...]) + bp_ref[...]
    m = jnp.mean(x2, axis=-1, keepdims=True)
    v = jnp.mean((x2 - m) ** 2, axis=-1, keepdims=True)
    y = (x2 - m) * jax.lax.rsqrt(v + 1e-5) * g2_ref[...] + b2_ref[...]
    y_ref[...] = y
    yb = y.astype(BF)
    logits = _dotf32(y, wr_ref[...]) + br_ref[...]
    mx = jnp.max(logits, axis=-1, keepdims=True)
    eg = jnp.exp(logits - mx)
    gates = eg / jnp.sum(eg, axis=-1, keepdims=True)
    iota = jax.lax.broadcasted_iota(jnp.int32, gates.shape, 1)
    v1 = jnp.max(gates, axis=-1, keepdims=True)
    i1 = jnp.min(jnp.where(gates == v1, iota, NUM_EXPERTS), axis=-1, keepdims=True)
    m1 = iota == i1
    g2nd = jnp.where(m1, -1.0, gates)
    v2 = jnp.max(g2nd, axis=-1, keepdims=True)
    i2 = jnp.min(jnp.where(g2nd == v2, iota, NUM_EXPERTS), axis=-1, keepdims=True)
    m2 = iota == i2
    denom = v1 + v2
    comb = (jnp.where(m1, v1, 0.0) + jnp.where(m2, v2, 0.0)) / denom
    _route(comb, dest_ref, w_ref, bexp_ref)
    hsh = _gelu(_dotf32(yb, ws1_ref[...]) + bs1_ref[...])
    shared = _dotf32(hsh.astype(BF), ws2_ref[...]) + bs2_ref[...]
    base_ref[...] = x2 + shared


def _k3(attn, xf, Wp_bf, b_proj, ln2_g, ln2_b, W_r, b_r, Ws1_bf, b_s1, Ws2_bf, b_s2):
    full = lambda *shape: pl.BlockSpec(shape, lambda: tuple(0 for _ in shape))
    return pl.pallas_call(
        _k3_body,
        grid=(),
        in_specs=[
            full(T, DIM), full(T, DIM), full(DIM, DIM), full(DIM),
            full(DIM), full(DIM), full(DIM, NUM_EXPERTS), full(NUM_EXPERTS),
            full(DIM, HID), full(HID), full(HID, DIM), full(DIM),
        ],
        out_specs=[full(T, DIM), full(T, DIM), full(8, T), full(8, T),
                   full(NBLK_PAD, 1)],
        out_shape=[
            jax.ShapeDtypeStruct((T, DIM), jnp.float32),
            jax.ShapeDtypeStruct((T, DIM), jnp.float32),
            jax.ShapeDtypeStruct((8, T), jnp.int32),
            jax.ShapeDtypeStruct((8, T), jnp.float32),
            jax.ShapeDtypeStruct((NBLK_PAD, 1), jnp.int32),
        ],
    )(attn, xf, Wp_bf, b_proj, ln2_g, ln2_b, W_r, b_r, Ws1_bf, b_s1, Ws2_bf, b_s2)


def _cumsum_lanes(x, n):
    d = 1
    while d < n:
        shifted = jnp.concatenate([jnp.zeros((x.shape[0], d), x.dtype), x[:, :n - d]],
                                  axis=1)
        x = x + shifted
        d *= 2
    return x


def _route(comb, dest_ref, w_ref, bexp_ref):
    c = jnp.transpose(comb, (1, 0))
    m = c > 0.0
    iot = jax.lax.broadcasted_iota(jnp.int32, (NUM_EXPERTS, T), 0)
    e_sm = jnp.min(jnp.where(m, iot, NUM_EXPERTS), axis=0, keepdims=True)
    e_bg = jnp.max(jnp.where(m, iot, -1), axis=0, keepdims=True)
    oh1 = (iot == e_sm).astype(jnp.float32)
    oh2 = (iot == e_bg).astype(jnp.float32)
    w1 = jnp.sum(c * oh1, axis=0, keepdims=True)
    w2 = jnp.sum(c * oh2, axis=0, keepdims=True)
    cs1 = _cumsum_lanes(oh1, T)
    cs2 = _cumsum_lanes(oh2, T)
    tot1 = cs1[:, T - 1:T]
    tot2 = cs2[:, T - 1:T]
    counts = tot1 + tot2
    nblk = jnp.floor((counts + (BLK - 1)) / BLK)
    le = (jax.lax.broadcasted_iota(jnp.int32, (NUM_EXPERTS, NUM_EXPERTS), 1)
          <= jax.lax.broadcasted_iota(jnp.int32, (NUM_EXPERTS, NUM_EXPERTS), 0))
    c_incl = _dotf32(le.astype(jnp.float32), nblk)
    offs = BLK * (c_incl - nblk)
    dest1 = jnp.sum((offs - 1.0) * oh1 + cs1 * oh1, axis=0, keepdims=True)
    dest2 = jnp.sum((offs + tot1 - 1.0) * oh2 + cs2 * oh2, axis=0, keepdims=True)
    zero6 = jnp.zeros((6, T), jnp.float32)
    dest_ref[...] = jnp.concatenate([dest1, dest2, zero6], axis=0).astype(jnp.int32)
    w_ref[...] = jnp.concatenate([w1, w2, zero6], axis=0)
    jot = jax.lax.broadcasted_iota(jnp.int32, (NBLK_PAD, NUM_EXPERTS), 0).astype(jnp.float32)
    cT = jnp.transpose(c_incl, (1, 0))
    bexp = jnp.sum((jot >= cT).astype(jnp.float32), axis=1, keepdims=True)
    bexp_ref[...] = jnp.clip(bexp, 0.0, NUM_EXPERTS - 1.0).astype(jnp.int32)


def _sc_dispatch(y32, dest32):
    mesh = plsc.VectorSubcoreMesh(core_axis_name="c", subcore_axis_name="s")

    @functools.partial(
        pl.kernel,
        out_type=jax.ShapeDtypeStruct((NBUF, DIM), jnp.float32),
        mesh=mesh,
        scratch_types=[
            pltpu.VMEM((CHUNK,), jnp.int32),
            pltpu.VMEM((CHUNK, DIM), jnp.float32),
            pltpu.SemaphoreType.DMA,
        ],
    )
    def disp(y_hbm, d_hbm, ybuf_hbm, idx_v, rows_v, sem):
        wid = jax.lax.axis_index("s") * 2 + jax.lax.axis_index("c")
        base = wid * CHUNK
        tok = jax.lax.rem(base, T)
        pltpu.sync_copy(d_hbm.at[wid], idx_v)
        pltpu.sync_copy(y_hbm.at[pl.ds(tok, CHUNK)], rows_v)
        pltpu.async_copy(rows_v, ybuf_hbm.at[idx_v], sem).wait()

    return disp(y32, dest32)


def _sc_combine(obuf, dest32):
    mesh = plsc.VectorSubcoreMesh(core_axis_name="c", subcore_axis_name="s")

    @functools.partial(
        pl.kernel,
        out_type=jax.ShapeDtypeStruct((SLOTS, DIM), jnp.float32),
        mesh=mesh,
        scratch_types=[
            pltpu.VMEM((CHUNK,), jnp.int32),
            pltpu.VMEM((CHUNK, DIM), jnp.float32),
            pltpu.SemaphoreType.DMA,
        ],
    )
    def comb_k(o_hbm, d_hbm, out_hbm, idx_v, rows_v, sem):
        wid = jax.lax.axis_index("s") * 2 + jax.lax.axis_index("c")
        base = wid * CHUNK
        pltpu.sync_copy(d_hbm.at[wid], idx_v)
        pltpu.async_copy(o_hbm.at[idx_v], rows_v, sem).wait()
        pltpu.sync_copy(rows_v, out_hbm.at[pl.ds(base, CHUNK)])

    return comb_k(obuf, dest32)


def _k6_body(bexp_ref, y_ref, w1_ref, b1_ref, w2_ref, b2_ref, o_ref):
    h = _gelu(_dotf32(y_ref[...].astype(BF), w1_ref[0]) + b1_ref[0])
    o_ref[...] = _dotf32(h.astype(BF), w2_ref[0]) + b2_ref[0]


def _k6(bexp, ybuf, We1_bf, b_e1r, We2_bf, b_e2r):
    grid_spec = pltpu.PrefetchScalarGridSpec(
        num_scalar_prefetch=1,
        grid=(NBLK,),
        in_specs=[
            pl.BlockSpec((BLK, DIM), lambda j, be: (j, 0)),
            pl.BlockSpec((1, DIM, HID), lambda j, be: (be[j], 0, 0)),
            pl.BlockSpec((1, 1, HID), lambda j, be: (be[j], 0, 0)),
            pl.BlockSpec((1, HID, DIM), lambda j, be: (be[j], 0, 0)),
            pl.BlockSpec((1, 1, DIM), lambda j, be: (be[j], 0, 0)),
        ],
        out_specs=pl.BlockSpec((BLK, DIM), lambda j, be: (j, 0)),
    )
    return pl.pallas_call(
        _k6_body,
        grid_spec=grid_spec,
        out_shape=jax.ShapeDtypeStruct((NBUF, DIM), jnp.float32),
    )(bexp, ybuf, We1_bf, b_e1r, We2_bf, b_e2r)


def _k5_body(base_ref, ot_ref, ob_ref, w_ref, o_ref):
    wp = jnp.transpose(w_ref[0:2, :], (1, 0))
    o_ref[...] = (base_ref[...] + wp[:, 0:1] * ot_ref[...]
                  + wp[:, 1:2] * ob_ref[...])


def _k5(base, out_slot, w8):
    blk = 512
    nb = T // blk
    return pl.pallas_call(
        _k5_body,
        grid=(nb,),
        in_specs=[
            pl.BlockSpec((blk, DIM), lambda i: (i, 0)),
            pl.BlockSpec((blk, DIM), lambda i: (i, 0)),
            pl.BlockSpec((blk, DIM), lambda i, _nb=nb: (i + _nb, 0)),
            pl.BlockSpec((8, blk), lambda i: (0, i)),
        ],
        out_specs=pl.BlockSpec((blk, DIM), lambda i: (i, 0)),
        out_shape=jax.ShapeDtypeStruct((T, DIM), jnp.float32),
    )(base, out_slot, out_slot, w8)


def kernel(x, ln1_g, ln1_b, ln2_g, ln2_b, W_qkv, b_qkv, rpb, W_proj, b_proj,
           W_r, b_r, W_e1, b_e1, W_e2, b_e2, W_s1, b_s1, W_s2, b_s2):
    xf = x.reshape(T, DIM)
    qkv = _k1(xf, ln1_g, ln1_b, W_qkv.astype(BF), b_qkv)
    qkv3 = qkv.reshape(B, H * W, 3 * DIM)
    bias = _make_bias(rpb)
    attn = _k2(bias, qkv3).reshape(T, DIM)
    y32, base, dest8, w8, bexp = _k3(
        attn, xf, W_proj.astype(BF), b_proj, ln2_g, ln2_b,
        W_r, b_r, W_s1.astype(BF), b_s1, W_s2.astype(BF), b_s2)
    dest32 = dest8[:2].reshape(NW, CHUNK)
    ybuf = _sc_dispatch(y32, dest32)
    obuf = _k6(bexp.reshape(NBLK_PAD), ybuf, W_e1.astype(BF),
               b_e1.reshape(NUM_EXPERTS, 1, HID), W_e2.astype(BF),
               b_e2.reshape(NUM_EXPERTS, 1, DIM))
    out_slot = _sc_combine(obuf, dest32)
    out = _k5(base, out_slot, w8)
    return out.reshape(B, H, W, DIM)
